# Initial kernel scaffold; baseline (speedup 1.0000x reference)
import jax
import jax.numpy as jnp
from jax.experimental import pallas as pl


def kernel(h, edge_index, WQ, bQ, WK, bK, WV, bV):
    raise NotImplementedError("write your pallas kernel here")


# speedup vs baseline: 23.8378x; 23.8378x over previous
import functools

import jax
import jax.numpy as jnp
from jax import lax
from jax.experimental import pallas as pl
from jax.experimental.pallas import tpu as pltpu
from jax.experimental.pallas import tpu_sc as plsc

N = 10000
E = 320000
IN_DIM = 128
H = 8
D = 16
HD = H * D
KV_W = 2 * HD
ACC_W = 144

NC = 2
NS = 16
NW = NC * NS
EW = E // NW
B = 64
NBLK = -(-EW // B)
EWP = NBLK * B
E_PAD = NW * EWP
PAD_DST = 10100
RPT = 632
N_PAD = NS * RPT
ZROWS = 640
ZPT = ZROWS // NS
CH = 128
NCHK = N_PAD // CH

_ROWS_PER_PROJ_BLOCK = 1000


def _proj_body(h_ref, wq_ref, wk_ref, wv_ref, bq_ref, bk_ref, bv_ref,
               q_ref, kv_ref):
    hb = h_ref[...]
    q = jnp.dot(hb, wq_ref[...], preferred_element_type=jnp.float32) + bq_ref[...]
    q_ref[...] = q * 0.25
    kv_ref[:, :HD] = jnp.dot(hb, wk_ref[...], preferred_element_type=jnp.float32) + bk_ref[...]
    kv_ref[:, HD:] = jnp.dot(hb, wv_ref[...], preferred_element_type=jnp.float32) + bv_ref[...]


_proj = pl.pallas_call(
    _proj_body,
    grid=(N // _ROWS_PER_PROJ_BLOCK,),
    in_specs=[
        pl.BlockSpec((_ROWS_PER_PROJ_BLOCK, IN_DIM), lambda i: (i, 0)),
        pl.BlockSpec((IN_DIM, HD), lambda i: (0, 0)),
        pl.BlockSpec((IN_DIM, HD), lambda i: (0, 0)),
        pl.BlockSpec((IN_DIM, HD), lambda i: (0, 0)),
        pl.BlockSpec((1, HD), lambda i: (0, 0)),
        pl.BlockSpec((1, HD), lambda i: (0, 0)),
        pl.BlockSpec((1, HD), lambda i: (0, 0)),
    ],
    out_specs=[
        pl.BlockSpec((_ROWS_PER_PROJ_BLOCK, HD), lambda i: (i, 0)),
        pl.BlockSpec((_ROWS_PER_PROJ_BLOCK, KV_W), lambda i: (i, 0)),
    ],
    out_shape=[
        jax.ShapeDtypeStruct((N, HD), jnp.float32),
        jax.ShapeDtypeStruct((N, KV_W), jnp.float32),
    ],
)

_mesh = plsc.VectorSubcoreMesh(core_axis_name="c", subcore_axis_name="s")
_sc_params = pltpu.CompilerParams(needs_layout_passes=False)


@functools.partial(
    pl.kernel,
    out_type=(
        jax.ShapeDtypeStruct((NC * N_PAD, HD), jnp.float32),
        jax.ShapeDtypeStruct((NC * ZROWS, HD), jnp.float32),
    ),
    mesh=_mesh,
    scratch_types=[
        pltpu.VMEM((B,), jnp.int32),
        pltpu.VMEM((B,), jnp.int32),
        pltpu.VMEM((B,), jnp.int32),
        pltpu.VMEM((B, KV_W), jnp.float32),
        pltpu.VMEM((B, HD), jnp.float32),
        pltpu.VMEM((B, HD), jnp.float32),
        pltpu.VMEM((B, HD), jnp.float32),
        pltpu.VMEM((2 * HD,), jnp.float32),
        pltpu.VMEM((B * H,), jnp.float32),
        pltpu.VMEM_SHARED((N_PAD, HD), jnp.float32),
        pltpu.VMEM_SHARED((ZROWS, HD), jnp.float32),
        pltpu.SemaphoreType.DMA,
    ],
    compiler_params=_sc_params,
)
def _edge_kernel(kv_hbm, q_hbm, src_hbm, dst_hbm, zv_hbm, zz_hbm,
                 outv_hbm, outz_hbm,
                 srcv, dstv, dzv, kvb, qb, msgv, msgz, tscr, wscr,
                 accv, accz, sem):
    c = lax.axis_index("c")
    s = lax.axis_index("s")
    wid = c * NS + s
    r0 = s * RPT
    z0 = s * ZPT
    pltpu.sync_copy(zv_hbm, accv.at[pl.ds(r0, RPT)])
    pltpu.sync_copy(zz_hbm, accz.at[pl.ds(z0, ZPT)])
    plsc.subcore_barrier()

    lane = lax.iota(jnp.int32, D)
    lane16 = lane * D
    lane8 = lane * H
    zvec = jnp.zeros((D,), jnp.float32)

    def zinit_body(e, carry):
        for kk in range(H):
            msgz[e, pl.ds(kk * D, D)] = zvec
        return carry

    lax.fori_loop(0, B, zinit_body, 0)

    ebase = wid * EWP

    def block_body(b, carry):
        off = ebase + b * B
        pltpu.sync_copy(src_hbm.at[pl.ds(off, B)], srcv)
        pltpu.sync_copy(dst_hbm.at[pl.ds(off, B)], dstv)
        cp1 = pltpu.async_copy(kv_hbm.at[srcv], kvb, sem)
        cp2 = pltpu.async_copy(q_hbm.at[dstv], qb, sem)
        cp1.wait()
        cp2.wait()

        def pair_body(p, carry2):
            e0 = 2 * p
            e1 = e0 + 1
            for eo, e in ((0, e0), (1, e1)):
                for h in range(H):
                    k = kvb[e, pl.ds(h * D, D)]
                    q = qb[e, pl.ds(h * D, D)]
                    tscr[pl.ds(eo * HD + h * D, D)] = k * q
            sums = jnp.zeros((D,), jnp.float32)
            for d in range(D):
                sums = sums + plsc.load_gather(tscr, [lane16 + d])
            sums = jnp.minimum(jnp.maximum(sums, -5.0), 5.0)
            w16 = jnp.exp(sums)
            wscr[pl.ds(p * D, D)] = w16
            for eo, e in ((0, e0), (1, e1)):
                for h in range(H):
                    w = jnp.full((D,), w16[eo * H + h], jnp.float32)
                    v = kvb[e, pl.ds(HD + h * D, D)]
                    msgv[e, pl.ds(h * D, D)] = w * v
            return carry2

        lax.fori_loop(0, B // 2, pair_body, 0)

        def zgrp_body(g, carry2):
            e16 = lane + g * D
            d16 = dstv[pl.ds(g * D, D)]
            dzv[pl.ds(g * D, D)] = lax.shift_right_logical(d16, 4)
            col = lax.shift_left(d16 & 15, 3)
            for h in range(H):
                wv = plsc.load_gather(wscr, [lane8 + (g * HD + h)])
                plsc.store_scatter(msgz, [e16, col + h], wv)
            return carry2

        lax.fori_loop(0, B // D, zgrp_body, 0)
        pltpu.sync_copy(msgv, accv.at[dstv], add=True)
        pltpu.sync_copy(msgz, accz.at[dzv], add=True)

        def zclr_body(g, carry2):
            e16 = lane + g * D
            d16 = dstv[pl.ds(g * D, D)]
            col = lax.shift_left(d16 & 15, 3)
            for h in range(H):
                plsc.store_scatter(msgz, [e16, col + h], zvec)
            return carry2

        lax.fori_loop(0, B // D, zclr_body, 0)
        return carry

    lax.fori_loop(0, NBLK, block_body, 0)
    plsc.subcore_barrier()
    pltpu.sync_copy(accv.at[pl.ds(r0, RPT)], outv_hbm.at[pl.ds(c * N_PAD + r0, RPT)])
    pltpu.sync_copy(accz.at[pl.ds(z0, ZPT)], outz_hbm.at[pl.ds(c * ZROWS + z0, ZPT)])


@functools.partial(
    pl.kernel,
    out_type=jax.ShapeDtypeStruct((N_PAD, HD), jnp.float32),
    mesh=_mesh,
    scratch_types=[
        pltpu.VMEM((CH, HD), jnp.float32),
        pltpu.VMEM((CH, HD), jnp.float32),
        pltpu.VMEM((CH // D, HD), jnp.float32),
        pltpu.VMEM((CH // D, HD), jnp.float32),
        pltpu.VMEM((CH, HD), jnp.float32),
    ],
    compiler_params=_sc_params,
)
def _combine_kernel(partv_hbm, partz_hbm, out_hbm, va, vb, za, zb, ob):
    c = lax.axis_index("c")
    s = lax.axis_index("s")
    wid = c * NS + s
    niter = (NCHK + NW - 1) // NW

    def iter_body(j, carry):
        chunk = wid + j * NW

        @pl.when(chunk < NCHK)
        def _():
            row = chunk * CH
            zrow = chunk * (CH // D)
            pltpu.sync_copy(partv_hbm.at[pl.ds(row, CH)], va)
            pltpu.sync_copy(partv_hbm.at[pl.ds(N_PAD + row, CH)], vb)
            pltpu.sync_copy(partz_hbm.at[pl.ds(zrow, CH // D)], za)
            pltpu.sync_copy(partz_hbm.at[pl.ds(ZROWS + zrow, CH // D)], zb)

            def row_body(r, carry2):
                zoff = lax.shift_left(lax.shift_right_logical(r, 4), 7) + \
                    lax.shift_left(r & 15, 3)
                zrow_i = lax.shift_right_logical(zoff, 7)
                zcol = zoff & 127
                zsum = za[zrow_i, pl.ds(zcol, D)] + zb[zrow_i, pl.ds(zcol, D)]
                for h in range(H):
                    wv = va[r, pl.ds(h * D, D)] + vb[r, pl.ds(h * D, D)]
                    ob[r, pl.ds(h * D, D)] = wv / jnp.full((D,), zsum[h], jnp.float32)
                return carry2

            lax.fori_loop(0, CH, row_body, 0)
            pltpu.sync_copy(ob, out_hbm.at[pl.ds(row, CH)])

        return carry

    lax.fori_loop(0, niter, iter_body, 0)


def kernel(h, edge_index, WQ, bQ, WK, bK, WV, bV):
    ei = edge_index.astype(jnp.int32)
    pad = E_PAD - E
    src = jnp.concatenate([ei[0], jnp.zeros((pad,), jnp.int32)])
    dst = jnp.concatenate([ei[1], jnp.full((pad,), PAD_DST, jnp.int32)])
    q, kv = _proj(h, WQ, WK, WV,
                  bQ.reshape(1, HD), bK.reshape(1, HD), bV.reshape(1, HD))
    zv = jnp.zeros((RPT, HD), jnp.float32)
    zz = jnp.zeros((ZPT, HD), jnp.float32)
    partv, partz = _edge_kernel(kv, q, src, dst, zv, zz)
    return _combine_kernel(partv, partz)[:N].reshape(N, H, D)
